# Initial kernel scaffold; baseline (speedup 1.0000x reference)
import jax
import jax.numpy as jnp
from jax.experimental import pallas as pl


def kernel(x_unsampled, x_sampled):
    raise NotImplementedError("write your pallas kernel here")


# speedup vs baseline: 26.2993x; 26.2993x over previous
import functools

import jax
import jax.numpy as jnp
from jax import lax
from jax.experimental import pallas as pl
from jax.experimental.pallas import tpu as pltpu
from jax.experimental.pallas import tpu_sc as plsc

B = 4
NQ = 8192
M = 2048
K = 3
L = 16
NC = 2
NS = 16
NW = NC * NS
WPB = NW // B
QPW = NQ // WPB
NG = QPW // L

_EPS2 = 1e-20
_BIGW = 1e10


def _bf16_round(x):
  u = lax.bitcast_convert_type(x, jnp.int32)
  r = (u + 0x7FFF + (lax.shift_right_logical(u, 16) & 1)) & jnp.int32(-65536)
  return lax.bitcast_convert_type(r, jnp.float32)


def _rsqrt_f32(x):
  i = lax.bitcast_convert_type(x, jnp.int32)
  i = 0x5F3759DF - lax.shift_right_logical(i, 1)
  y = lax.bitcast_convert_type(i, jnp.float32)
  for _ in range(3):
    y = y * (1.5 - 0.5 * x * y * y)
  return y


def _tec_body(qx_h, qy_h, qz_h, kx_h, ky_h, kz_h,
              i0_h, i1_h, i2_h, w0_h, w1_h, w2_h,
              qx_v, qy_v, qz_v, kx_v, ky_v, kz_v, h_v,
              i0_s, i1_s, i2_s, w0_s, w1_s, w2_s):
  wid = lax.axis_index("s") * NC + lax.axis_index("c")
  b = wid // WPB
  q0 = (wid % WPB) * QPW

  pltpu.sync_copy(qx_h.at[b, pl.ds(q0, QPW)], qx_v)
  pltpu.sync_copy(qy_h.at[b, pl.ds(q0, QPW)], qy_v)
  pltpu.sync_copy(qz_h.at[b, pl.ds(q0, QPW)], qz_v)
  pltpu.sync_copy(kx_h.at[b], kx_v)
  pltpu.sync_copy(ky_h.at[b], ky_v)
  pltpu.sync_copy(kz_h.at[b], kz_v)

  def h_body(i, carry):
    sl = pl.ds(i * L, L)
    kx = kx_v[sl]
    ky = ky_v[sl]
    kz = kz_v[sl]
    h_v[sl] = kx * kx + ky * ky + kz * kz
    kx_v[sl] = _bf16_round(kx)
    ky_v[sl] = _bf16_round(ky)
    kz_v[sl] = _bf16_round(kz)
    return carry
  lax.fori_loop(0, M // L, h_body, 0)

  lanes = lax.iota(jnp.int32, L)
  inf_v = jnp.full((L,), jnp.inf, jnp.float32)
  zero_i = jnp.zeros((L,), jnp.int32)

  def g_body(g, carry):
    qx = qx_v[pl.ds(g * L, L)]
    qy = qy_v[pl.ds(g * L, L)]
    qz = qz_v[pl.ds(g * L, L)]
    qxr = _bf16_round(qx)
    qyr = _bf16_round(qy)
    qzr = _bf16_round(qz)
    t1 = qx * qx + qy * qy + qz * qz

    def jb_body(jb, c):
      s0, s1, s2, i0, i1, i2 = c
      kxv = kx_v[pl.ds(jb * L, L)]
      kyv = ky_v[pl.ds(jb * L, L)]
      kzv = kz_v[pl.ds(jb * L, L)]
      hv = h_v[pl.ds(jb * L, L)]
      jbase = jb * L
      for m in range(L):
        t3 = qxr * kxv[m] + qyr * kyv[m] + qzr * kzv[m]
        s = jnp.maximum((t1 + hv[m]) - 2.0 * t3, 0.0)
        jv = zero_i + (jbase + m)
        c0 = s < s0
        c1 = s < s1
        c2 = s < s2
        n1 = jnp.where(c0, s0, jnp.where(c1, s, s1))
        m1 = jnp.where(c0, i0, jnp.where(c1, jv, i1))
        s2 = jnp.where(c1, s1, jnp.where(c2, s, s2))
        i2 = jnp.where(c1, i1, jnp.where(c2, jv, i2))
        s0 = jnp.where(c0, s, s0)
        i0 = jnp.where(c0, jv, i0)
        s1, i1 = n1, m1
      return (s0, s1, s2, i0, i1, i2)

    s0, s1, s2, i0, i1, i2 = lax.fori_loop(
        0, M // L, jb_body, (inf_v, inf_v, inf_v, zero_i, zero_i, zero_i))

    ws = []
    for sm in (s0, s1, s2):
      d2 = jnp.maximum(sm, 0.0)
      w = jnp.where(d2 < _EPS2, _BIGW, _rsqrt_f32(d2))
      ws.append(w)
    wsum = ws[0] + ws[1] + ws[2]

    sl = pl.ds(g * L, L)
    i0_s[sl] = i0
    i1_s[sl] = i1
    i2_s[sl] = i2
    w0_s[sl] = ws[0] / wsum
    w1_s[sl] = ws[1] / wsum
    w2_s[sl] = ws[2] / wsum
    return carry

  lax.fori_loop(0, NG, g_body, 0)

  sl = pl.ds(q0, QPW)
  pltpu.sync_copy(i0_s, i0_h.at[b, sl])
  pltpu.sync_copy(i1_s, i1_h.at[b, sl])
  pltpu.sync_copy(i2_s, i2_h.at[b, sl])
  pltpu.sync_copy(w0_s, w0_h.at[b, sl])
  pltpu.sync_copy(w1_s, w1_h.at[b, sl])
  pltpu.sync_copy(w2_s, w2_h.at[b, sl])


_mesh = plsc.VectorSubcoreMesh(core_axis_name="c", subcore_axis_name="s")

_sc_knn = functools.partial(
    pl.kernel,
    mesh=_mesh,
    out_type=[jax.ShapeDtypeStruct((B, NQ), jnp.int32)] * 3
    + [jax.ShapeDtypeStruct((B, NQ), jnp.float32)] * 3,
    scratch_types=[
        pltpu.VMEM((QPW,), jnp.float32),
        pltpu.VMEM((QPW,), jnp.float32),
        pltpu.VMEM((QPW,), jnp.float32),
        pltpu.VMEM((M,), jnp.float32),
        pltpu.VMEM((M,), jnp.float32),
        pltpu.VMEM((M,), jnp.float32),
        pltpu.VMEM((M,), jnp.float32),
        pltpu.VMEM((QPW,), jnp.int32),
        pltpu.VMEM((QPW,), jnp.int32),
        pltpu.VMEM((QPW,), jnp.int32),
        pltpu.VMEM((QPW,), jnp.float32),
        pltpu.VMEM((QPW,), jnp.float32),
        pltpu.VMEM((QPW,), jnp.float32),
    ],
)(_tec_body)


@jax.jit
def kernel(x_unsampled, x_sampled):
  qx = x_unsampled[:, :, 0]
  qy = x_unsampled[:, :, 1]
  qz = x_unsampled[:, :, 2]
  kx = x_sampled[:, :, 0]
  ky = x_sampled[:, :, 1]
  kz = x_sampled[:, :, 2]
  ia, ib, ic, wa, wb, wc = _sc_knn(qx, qy, qz, kx, ky, kz)
  idx = jnp.stack((ia, ib, ic), axis=-1)
  w = jnp.stack((wa, wb, wc), axis=-1)[..., None]
  return idx, w


# speedup vs baseline: 27.1634x; 1.0329x over previous
import functools

import jax
import jax.numpy as jnp
from jax import lax
from jax.experimental import pallas as pl
from jax.experimental.pallas import tpu as pltpu
from jax.experimental.pallas import tpu_sc as plsc

B = 4
NQ = 8192
M = 2048
K = 3
L = 16
NC = 2
NS = 16
NW = NC * NS
WPB = NW // B
QPW = NQ // WPB
NG = QPW // L

_EPS2 = 1e-20
_BIGW = 1e10


def _bf16_round(x):
  u = lax.bitcast_convert_type(x, jnp.int32)
  r = (u + 0x7FFF + (lax.shift_right_logical(u, 16) & 1)) & jnp.int32(-65536)
  return lax.bitcast_convert_type(r, jnp.float32)


def _rsqrt_f32(x):
  i = lax.bitcast_convert_type(x, jnp.int32)
  i = 0x5F3759DF - lax.shift_right_logical(i, 1)
  y = lax.bitcast_convert_type(i, jnp.float32)
  for _ in range(3):
    y = y * (1.5 - 0.5 * x * y * y)
  return y


def _tec_body(qx_h, qy_h, qz_h, kx_h, ky_h, kz_h,
              i0_h, i1_h, i2_h, w0_h, w1_h, w2_h,
              qx_v, qy_v, qz_v, kx_v, ky_v, kz_v, h_v,
              i0_s, i1_s, i2_s, w0_s, w1_s, w2_s):
  wid = lax.axis_index("s") * NC + lax.axis_index("c")
  b = wid // WPB
  q0 = (wid % WPB) * QPW

  pltpu.sync_copy(qx_h.at[b, pl.ds(q0, QPW)], qx_v)
  pltpu.sync_copy(qy_h.at[b, pl.ds(q0, QPW)], qy_v)
  pltpu.sync_copy(qz_h.at[b, pl.ds(q0, QPW)], qz_v)
  pltpu.sync_copy(kx_h.at[b], kx_v)
  pltpu.sync_copy(ky_h.at[b], ky_v)
  pltpu.sync_copy(kz_h.at[b], kz_v)

  def h_body(i, carry):
    sl = pl.ds(i * L, L)
    kx = kx_v[sl]
    ky = ky_v[sl]
    kz = kz_v[sl]
    h_v[sl] = kx * kx + ky * ky + kz * kz
    kx_v[sl] = 2.0 * _bf16_round(kx)
    ky_v[sl] = 2.0 * _bf16_round(ky)
    kz_v[sl] = 2.0 * _bf16_round(kz)
    return carry
  lax.fori_loop(0, M // L, h_body, 0)

  lanes = lax.iota(jnp.int32, L)
  inf_v = jnp.full((L,), jnp.inf, jnp.float32)
  zero_i = jnp.zeros((L,), jnp.int32)

  def g_body(g, carry):
    qx = qx_v[pl.ds(g * L, L)]
    qy = qy_v[pl.ds(g * L, L)]
    qz = qz_v[pl.ds(g * L, L)]
    qxr = _bf16_round(qx)
    qyr = _bf16_round(qy)
    qzr = _bf16_round(qz)
    t1 = qx * qx + qy * qy + qz * qz

    def jb_body(jb, c):
      s0, s1, s2, i0, i1, i2 = c
      kxv = kx_v[pl.ds(jb * L, L)]
      kyv = ky_v[pl.ds(jb * L, L)]
      kzv = kz_v[pl.ds(jb * L, L)]
      hv = h_v[pl.ds(jb * L, L)]
      jbase = jb * L
      for m in range(L):
        t3d = qxr * kxv[m] + qyr * kyv[m] + qzr * kzv[m]
        s = jnp.maximum((t1 + hv[m]) - t3d, 0.0)
        jv = zero_i + (jbase + m)
        c0 = s < s0
        c1 = s < s1
        c2 = s < s2
        n1 = jnp.where(c0, s0, jnp.minimum(s, s1))
        m1 = jnp.where(c0, i0, jnp.where(c1, jv, i1))
        s2 = jnp.where(c1, s1, jnp.minimum(s, s2))
        i2 = jnp.where(c1, i1, jnp.where(c2, jv, i2))
        s0 = jnp.minimum(s, s0)
        i0 = jnp.where(c0, jv, i0)
        s1, i1 = n1, m1
      return (s0, s1, s2, i0, i1, i2)

    s0, s1, s2, i0, i1, i2 = lax.fori_loop(
        0, M // L, jb_body, (inf_v, inf_v, inf_v, zero_i, zero_i, zero_i))

    ws = []
    for sm in (s0, s1, s2):
      d2 = jnp.maximum(sm, 0.0)
      w = jnp.where(d2 < _EPS2, _BIGW, _rsqrt_f32(d2))
      ws.append(w)
    wsum = ws[0] + ws[1] + ws[2]

    sl = pl.ds(g * L, L)
    i0_s[sl] = i0
    i1_s[sl] = i1
    i2_s[sl] = i2
    w0_s[sl] = ws[0] / wsum
    w1_s[sl] = ws[1] / wsum
    w2_s[sl] = ws[2] / wsum
    return carry

  lax.fori_loop(0, NG, g_body, 0)

  sl = pl.ds(q0, QPW)
  pltpu.sync_copy(i0_s, i0_h.at[b, sl])
  pltpu.sync_copy(i1_s, i1_h.at[b, sl])
  pltpu.sync_copy(i2_s, i2_h.at[b, sl])
  pltpu.sync_copy(w0_s, w0_h.at[b, sl])
  pltpu.sync_copy(w1_s, w1_h.at[b, sl])
  pltpu.sync_copy(w2_s, w2_h.at[b, sl])


_mesh = plsc.VectorSubcoreMesh(core_axis_name="c", subcore_axis_name="s")

_sc_knn = functools.partial(
    pl.kernel,
    mesh=_mesh,
    out_type=[jax.ShapeDtypeStruct((B, NQ), jnp.int32)] * 3
    + [jax.ShapeDtypeStruct((B, NQ), jnp.float32)] * 3,
    scratch_types=[
        pltpu.VMEM((QPW,), jnp.float32),
        pltpu.VMEM((QPW,), jnp.float32),
        pltpu.VMEM((QPW,), jnp.float32),
        pltpu.VMEM((M,), jnp.float32),
        pltpu.VMEM((M,), jnp.float32),
        pltpu.VMEM((M,), jnp.float32),
        pltpu.VMEM((M,), jnp.float32),
        pltpu.VMEM((QPW,), jnp.int32),
        pltpu.VMEM((QPW,), jnp.int32),
        pltpu.VMEM((QPW,), jnp.int32),
        pltpu.VMEM((QPW,), jnp.float32),
        pltpu.VMEM((QPW,), jnp.float32),
        pltpu.VMEM((QPW,), jnp.float32),
    ],
)(_tec_body)


@jax.jit
def kernel(x_unsampled, x_sampled):
  qx = x_unsampled[:, :, 0]
  qy = x_unsampled[:, :, 1]
  qz = x_unsampled[:, :, 2]
  kx = x_sampled[:, :, 0]
  ky = x_sampled[:, :, 1]
  kz = x_sampled[:, :, 2]
  ia, ib, ic, wa, wb, wc = _sc_knn(qx, qy, qz, kx, ky, kz)
  idx = jnp.stack((ia, ib, ic), axis=-1)
  w = jnp.stack((wa, wb, wc), axis=-1)[..., None]
  return idx, w


# speedup vs baseline: 51.1797x; 1.8841x over previous
import functools

import jax
import jax.numpy as jnp
from jax import lax
from jax.experimental import pallas as pl
from jax.experimental.pallas import tpu as pltpu
from jax.experimental.pallas import tpu_sc as plsc

B = 4
NQ = 8192
M = 2048
K = 3
L = 16
NC = 2
NS = 16
NW = NC * NS
WPB = NW // B

NQ_SC = 4096
NQ_TC = NQ - NQ_SC
TQ = 256
NGT = NQ_TC // TQ

QPW = NQ_SC // WPB
NG = QPW // L

_EPS2 = 1e-20
_BIGW = 1e10


def _bf16_round(x):
  u = lax.bitcast_convert_type(x, jnp.int32)
  r = (u + 0x7FFF + (lax.shift_right_logical(u, 16) & 1)) & jnp.int32(-65536)
  return lax.bitcast_convert_type(r, jnp.float32)


def _rsqrt_f32(x):
  i = lax.bitcast_convert_type(x, jnp.int32)
  i = 0x5F3759DF - lax.shift_right_logical(i, 1)
  y = lax.bitcast_convert_type(i, jnp.float32)
  for _ in range(3):
    y = y * (1.5 - 0.5 * x * y * y)
  return y


def _tec_body(qx_h, qy_h, qz_h, kx_h, ky_h, kz_h,
              i0_h, i1_h, i2_h, w0_h, w1_h, w2_h,
              qx_v, qy_v, qz_v, kx_v, ky_v, kz_v, h_v,
              i0_s, i1_s, i2_s, w0_s, w1_s, w2_s):
  wid = lax.axis_index("s") * NC + lax.axis_index("c")
  b = wid // WPB
  q0 = (wid % WPB) * QPW

  pltpu.sync_copy(qx_h.at[b, pl.ds(q0, QPW)], qx_v)
  pltpu.sync_copy(qy_h.at[b, pl.ds(q0, QPW)], qy_v)
  pltpu.sync_copy(qz_h.at[b, pl.ds(q0, QPW)], qz_v)
  pltpu.sync_copy(kx_h.at[b], kx_v)
  pltpu.sync_copy(ky_h.at[b], ky_v)
  pltpu.sync_copy(kz_h.at[b], kz_v)

  def h_body(i, carry):
    sl = pl.ds(i * L, L)
    kx = kx_v[sl]
    ky = ky_v[sl]
    kz = kz_v[sl]
    h_v[sl] = kx * kx + ky * ky + kz * kz
    kx_v[sl] = 2.0 * _bf16_round(kx)
    ky_v[sl] = 2.0 * _bf16_round(ky)
    kz_v[sl] = 2.0 * _bf16_round(kz)
    return carry
  lax.fori_loop(0, M // L, h_body, 0)

  lanes = lax.iota(jnp.int32, L)
  inf_v = jnp.full((L,), jnp.inf, jnp.float32)
  zero_i = jnp.zeros((L,), jnp.int32)

  def g_body(g, carry):
    qx = qx_v[pl.ds(g * L, L)]
    qy = qy_v[pl.ds(g * L, L)]
    qz = qz_v[pl.ds(g * L, L)]
    qxr = _bf16_round(qx)
    qyr = _bf16_round(qy)
    qzr = _bf16_round(qz)
    t1 = qx * qx + qy * qy + qz * qz

    def jb_body(jb, c):
      s0, s1, s2, i0, i1, i2 = c
      kxv = kx_v[pl.ds(jb * L, L)]
      kyv = ky_v[pl.ds(jb * L, L)]
      kzv = kz_v[pl.ds(jb * L, L)]
      hv = h_v[pl.ds(jb * L, L)]
      jbase = jb * L
      for m in range(L):
        t3d = qxr * kxv[m] + qyr * kyv[m] + qzr * kzv[m]
        s = jnp.maximum((t1 + hv[m]) - t3d, 0.0)
        jv = zero_i + (jbase + m)
        c0 = s < s0
        c1 = s < s1
        c2 = s < s2
        n1 = jnp.where(c0, s0, jnp.minimum(s, s1))
        m1 = jnp.where(c0, i0, jnp.where(c1, jv, i1))
        s2 = jnp.where(c1, s1, jnp.minimum(s, s2))
        i2 = jnp.where(c1, i1, jnp.where(c2, jv, i2))
        s0 = jnp.minimum(s, s0)
        i0 = jnp.where(c0, jv, i0)
        s1, i1 = n1, m1
      return (s0, s1, s2, i0, i1, i2)

    s0, s1, s2, i0, i1, i2 = lax.fori_loop(
        0, M // L, jb_body, (inf_v, inf_v, inf_v, zero_i, zero_i, zero_i))

    ws = []
    for sm in (s0, s1, s2):
      d2 = jnp.maximum(sm, 0.0)
      w = jnp.where(d2 < _EPS2, _BIGW, _rsqrt_f32(d2))
      ws.append(w)
    wsum = ws[0] + ws[1] + ws[2]

    sl = pl.ds(g * L, L)
    i0_s[sl] = i0
    i1_s[sl] = i1
    i2_s[sl] = i2
    w0_s[sl] = ws[0] / wsum
    w1_s[sl] = ws[1] / wsum
    w2_s[sl] = ws[2] / wsum
    return carry

  lax.fori_loop(0, NG, g_body, 0)

  sl = pl.ds(q0, QPW)
  pltpu.sync_copy(i0_s, i0_h.at[b, sl])
  pltpu.sync_copy(i1_s, i1_h.at[b, sl])
  pltpu.sync_copy(i2_s, i2_h.at[b, sl])
  pltpu.sync_copy(w0_s, w0_h.at[b, sl])
  pltpu.sync_copy(w1_s, w1_h.at[b, sl])
  pltpu.sync_copy(w2_s, w2_h.at[b, sl])


_mesh = plsc.VectorSubcoreMesh(core_axis_name="c", subcore_axis_name="s")

_sc_knn = functools.partial(
    pl.kernel,
    mesh=_mesh,
    out_type=[jax.ShapeDtypeStruct((B, NQ_SC), jnp.int32)] * 3
    + [jax.ShapeDtypeStruct((B, NQ_SC), jnp.float32)] * 3,
    scratch_types=[
        pltpu.VMEM((QPW,), jnp.float32),
        pltpu.VMEM((QPW,), jnp.float32),
        pltpu.VMEM((QPW,), jnp.float32),
        pltpu.VMEM((M,), jnp.float32),
        pltpu.VMEM((M,), jnp.float32),
        pltpu.VMEM((M,), jnp.float32),
        pltpu.VMEM((M,), jnp.float32),
        pltpu.VMEM((QPW,), jnp.int32),
        pltpu.VMEM((QPW,), jnp.int32),
        pltpu.VMEM((QPW,), jnp.int32),
        pltpu.VMEM((QPW,), jnp.float32),
        pltpu.VMEM((QPW,), jnp.float32),
        pltpu.VMEM((QPW,), jnp.float32),
    ],
)(_tec_body)


def _tc_body(qx_r, qy_r, qz_r, kx_r, ky_r, kz_r,
             i0_r, i1_r, i2_r, w0_r, w1_r, w2_r):
  qx = qx_r[0, 0]
  qy = qy_r[0, 0]
  qz = qz_r[0, 0]
  kx = kx_r[0]
  ky = ky_r[0]
  kz = kz_r[0]

  t1 = qx * qx + qy * qy + qz * qz
  t2 = kx * kx + ky * ky + kz * kz
  qxr = _bf16_round(qx)
  qyr = _bf16_round(qy)
  qzr = _bf16_round(qz)
  kxr = 2.0 * _bf16_round(kx)
  kyr = 2.0 * _bf16_round(ky)
  kzr = 2.0 * _bf16_round(kz)

  t3d = qxr * kxr + qyr * kyr + qzr * kzr
  s = jnp.maximum((t1 + t2) - t3d, 0.0)
  iota = lax.broadcasted_iota(jnp.int32, (M, TQ), 0)
  inf = jnp.float32(jnp.inf)

  idxs, d2s = [], []
  for r in range(K):
    mn = jnp.min(s, axis=0, keepdims=True)
    ir = jnp.min(jnp.where(s == mn, iota, M), axis=0, keepdims=True)
    idxs.append(ir)
    d2s.append(mn)
    if r < K - 1:
      s = jnp.where(iota == ir, inf, s)

  ws = [jnp.where(d2 < _EPS2, _BIGW, 1.0 / jnp.sqrt(d2)) for d2 in d2s]
  wsum = ws[0] + ws[1] + ws[2]

  i0_r[0, 0] = idxs[0]
  i1_r[0, 0] = idxs[1]
  i2_r[0, 0] = idxs[2]
  w0_r[0, 0] = ws[0] / wsum
  w1_r[0, 0] = ws[1] / wsum
  w2_r[0, 0] = ws[2] / wsum


_q_spec = pl.BlockSpec((1, 1, 1, TQ), lambda b, t: (b, t, 0, 0))
_k_spec = pl.BlockSpec((1, M, 1), lambda b, t: (b, 0, 0))
_o_spec = pl.BlockSpec((1, 1, 1, TQ), lambda b, t: (b, t, 0, 0))

_tc_knn = pl.pallas_call(
    _tc_body,
    grid=(B, NGT),
    in_specs=[_q_spec] * 3 + [_k_spec] * 3,
    out_specs=[_o_spec] * 6,
    out_shape=[jax.ShapeDtypeStruct((B, NGT, 1, TQ), jnp.int32)] * 3
    + [jax.ShapeDtypeStruct((B, NGT, 1, TQ), jnp.float32)] * 3,
)


@jax.jit
def kernel(x_unsampled, x_sampled):
  qx = x_unsampled[:, :, 0]
  qy = x_unsampled[:, :, 1]
  qz = x_unsampled[:, :, 2]
  kx = x_sampled[:, :, 0]
  ky = x_sampled[:, :, 1]
  kz = x_sampled[:, :, 2]

  sc_out = _sc_knn(qx[:, :NQ_SC], qy[:, :NQ_SC], qz[:, :NQ_SC], kx, ky, kz)

  def tile4(a):
    return a[:, NQ_SC:].reshape(B, NGT, 1, TQ)
  kcol = lambda a: a.reshape(B, M, 1)
  tc_out = _tc_knn(tile4(qx), tile4(qy), tile4(qz), kcol(kx), kcol(ky),
                   kcol(kz))

  parts = []
  for p_sc, p_tc in zip(sc_out, tc_out):
    parts.append(jnp.concatenate((p_sc, p_tc.reshape(B, NQ_TC)), axis=1))
  ia, ib, ic, wa, wb, wc = parts
  idx = jnp.stack((ia, ib, ic), axis=-1)
  w = jnp.stack((wa, wb, wc), axis=-1)[..., None]
  return idx, w


# speedup vs baseline: 55.5840x; 1.0861x over previous
import functools

import jax
import jax.numpy as jnp
from jax import lax
from jax.experimental import pallas as pl
from jax.experimental.pallas import tpu as pltpu
from jax.experimental.pallas import tpu_sc as plsc

B = 4
NQ = 8192
M = 2048
K = 3
L = 16
NC = 2
NS = 16
NW = NC * NS
WPB = NW // B

NQ_SC = 2048
NQ_TC = NQ - NQ_SC
TQ = 256
NGT = NQ_TC // TQ

QPW = NQ_SC // WPB
NG = QPW // L

_EPS2 = 1e-20
_BIGW = 1e10


def _bf16_round(x):
  u = lax.bitcast_convert_type(x, jnp.int32)
  r = (u + 0x7FFF + (lax.shift_right_logical(u, 16) & 1)) & jnp.int32(-65536)
  return lax.bitcast_convert_type(r, jnp.float32)


def _rsqrt_f32(x):
  i = lax.bitcast_convert_type(x, jnp.int32)
  i = 0x5F3759DF - lax.shift_right_logical(i, 1)
  y = lax.bitcast_convert_type(i, jnp.float32)
  for _ in range(3):
    y = y * (1.5 - 0.5 * x * y * y)
  return y


def _tec_body(qx_h, qy_h, qz_h, kx_h, ky_h, kz_h,
              i0_h, i1_h, i2_h, w0_h, w1_h, w2_h,
              qx_v, qy_v, qz_v, kx_v, ky_v, kz_v, h_v,
              i0_s, i1_s, i2_s, w0_s, w1_s, w2_s):
  wid = lax.axis_index("s") * NC + lax.axis_index("c")
  b = wid // WPB
  q0 = (wid % WPB) * QPW

  pltpu.sync_copy(qx_h.at[b, pl.ds(q0, QPW)], qx_v)
  pltpu.sync_copy(qy_h.at[b, pl.ds(q0, QPW)], qy_v)
  pltpu.sync_copy(qz_h.at[b, pl.ds(q0, QPW)], qz_v)
  pltpu.sync_copy(kx_h.at[b], kx_v)
  pltpu.sync_copy(ky_h.at[b], ky_v)
  pltpu.sync_copy(kz_h.at[b], kz_v)

  def h_body(i, carry):
    sl = pl.ds(i * L, L)
    kx = kx_v[sl]
    ky = ky_v[sl]
    kz = kz_v[sl]
    h_v[sl] = kx * kx + ky * ky + kz * kz
    kx_v[sl] = 2.0 * _bf16_round(kx)
    ky_v[sl] = 2.0 * _bf16_round(ky)
    kz_v[sl] = 2.0 * _bf16_round(kz)
    return carry
  lax.fori_loop(0, M // L, h_body, 0)

  lanes = lax.iota(jnp.int32, L)
  inf_v = jnp.full((L,), jnp.inf, jnp.float32)
  zero_i = jnp.zeros((L,), jnp.int32)

  def g_body(g, carry):
    qx = qx_v[pl.ds(g * L, L)]
    qy = qy_v[pl.ds(g * L, L)]
    qz = qz_v[pl.ds(g * L, L)]
    qxr = _bf16_round(qx)
    qyr = _bf16_round(qy)
    qzr = _bf16_round(qz)
    t1 = qx * qx + qy * qy + qz * qz

    def jb_body(jb, c):
      s0, s1, s2, i0, i1, i2 = c
      kxv = kx_v[pl.ds(jb * L, L)]
      kyv = ky_v[pl.ds(jb * L, L)]
      kzv = kz_v[pl.ds(jb * L, L)]
      hv = h_v[pl.ds(jb * L, L)]
      jbase = jb * L
      for m in range(L):
        t3d = qxr * kxv[m] + qyr * kyv[m] + qzr * kzv[m]
        s = jnp.maximum((t1 + hv[m]) - t3d, 0.0)
        jv = zero_i + (jbase + m)
        c0 = s < s0
        c1 = s < s1
        c2 = s < s2
        n1 = jnp.where(c0, s0, jnp.minimum(s, s1))
        m1 = jnp.where(c0, i0, jnp.where(c1, jv, i1))
        s2 = jnp.where(c1, s1, jnp.minimum(s, s2))
        i2 = jnp.where(c1, i1, jnp.where(c2, jv, i2))
        s0 = jnp.minimum(s, s0)
        i0 = jnp.where(c0, jv, i0)
        s1, i1 = n1, m1
      return (s0, s1, s2, i0, i1, i2)

    s0, s1, s2, i0, i1, i2 = lax.fori_loop(
        0, M // L, jb_body, (inf_v, inf_v, inf_v, zero_i, zero_i, zero_i))

    ws = []
    for sm in (s0, s1, s2):
      d2 = jnp.maximum(sm, 0.0)
      w = jnp.where(d2 < _EPS2, _BIGW, _rsqrt_f32(d2))
      ws.append(w)
    wsum = ws[0] + ws[1] + ws[2]

    sl = pl.ds(g * L, L)
    i0_s[sl] = i0
    i1_s[sl] = i1
    i2_s[sl] = i2
    w0_s[sl] = ws[0] / wsum
    w1_s[sl] = ws[1] / wsum
    w2_s[sl] = ws[2] / wsum
    return carry

  lax.fori_loop(0, NG, g_body, 0)

  sl = pl.ds(q0, QPW)
  pltpu.sync_copy(i0_s, i0_h.at[b, sl])
  pltpu.sync_copy(i1_s, i1_h.at[b, sl])
  pltpu.sync_copy(i2_s, i2_h.at[b, sl])
  pltpu.sync_copy(w0_s, w0_h.at[b, sl])
  pltpu.sync_copy(w1_s, w1_h.at[b, sl])
  pltpu.sync_copy(w2_s, w2_h.at[b, sl])


_mesh = plsc.VectorSubcoreMesh(core_axis_name="c", subcore_axis_name="s")

_sc_knn = functools.partial(
    pl.kernel,
    mesh=_mesh,
    out_type=[jax.ShapeDtypeStruct((B, NQ_SC), jnp.int32)] * 3
    + [jax.ShapeDtypeStruct((B, NQ_SC), jnp.float32)] * 3,
    scratch_types=[
        pltpu.VMEM((QPW,), jnp.float32),
        pltpu.VMEM((QPW,), jnp.float32),
        pltpu.VMEM((QPW,), jnp.float32),
        pltpu.VMEM((M,), jnp.float32),
        pltpu.VMEM((M,), jnp.float32),
        pltpu.VMEM((M,), jnp.float32),
        pltpu.VMEM((M,), jnp.float32),
        pltpu.VMEM((QPW,), jnp.int32),
        pltpu.VMEM((QPW,), jnp.int32),
        pltpu.VMEM((QPW,), jnp.int32),
        pltpu.VMEM((QPW,), jnp.float32),
        pltpu.VMEM((QPW,), jnp.float32),
        pltpu.VMEM((QPW,), jnp.float32),
    ],
)(_tec_body)


def _tc_body(qx_r, qy_r, qz_r, kx_r, ky_r, kz_r,
             i0_r, i1_r, i2_r, w0_r, w1_r, w2_r):
  qx = qx_r[0, 0]
  qy = qy_r[0, 0]
  qz = qz_r[0, 0]
  kx = kx_r[0]
  ky = ky_r[0]
  kz = kz_r[0]

  t1 = qx * qx + qy * qy + qz * qz
  t2 = kx * kx + ky * ky + kz * kz
  qxr = _bf16_round(qx)
  qyr = _bf16_round(qy)
  qzr = _bf16_round(qz)
  kxr = 2.0 * _bf16_round(kx)
  kyr = 2.0 * _bf16_round(ky)
  kzr = 2.0 * _bf16_round(kz)

  t3d = qxr * kxr + qyr * kyr + qzr * kzr
  s = jnp.maximum((t1 + t2) - t3d, 0.0)
  iota = lax.broadcasted_iota(jnp.int32, (M, TQ), 0)
  inf = jnp.float32(jnp.inf)

  idxs, d2s = [], []
  for r in range(K):
    mn = jnp.min(s, axis=0, keepdims=True)
    ir = jnp.min(jnp.where(s == mn, iota, M), axis=0, keepdims=True)
    idxs.append(ir)
    d2s.append(mn)
    if r < K - 1:
      s = jnp.where(iota == ir, inf, s)

  ws = [jnp.where(d2 < _EPS2, _BIGW, 1.0 / jnp.sqrt(d2)) for d2 in d2s]
  wsum = ws[0] + ws[1] + ws[2]

  i0_r[0, 0] = idxs[0]
  i1_r[0, 0] = idxs[1]
  i2_r[0, 0] = idxs[2]
  w0_r[0, 0] = ws[0] / wsum
  w1_r[0, 0] = ws[1] / wsum
  w2_r[0, 0] = ws[2] / wsum


_q_spec = pl.BlockSpec((1, 1, 1, TQ), lambda b, t: (b, t, 0, 0))
_k_spec = pl.BlockSpec((1, M, 1), lambda b, t: (b, 0, 0))
_o_spec = pl.BlockSpec((1, 1, 1, TQ), lambda b, t: (b, t, 0, 0))

_tc_knn = pl.pallas_call(
    _tc_body,
    grid=(B, NGT),
    in_specs=[_q_spec] * 3 + [_k_spec] * 3,
    out_specs=[_o_spec] * 6,
    out_shape=[jax.ShapeDtypeStruct((B, NGT, 1, TQ), jnp.int32)] * 3
    + [jax.ShapeDtypeStruct((B, NGT, 1, TQ), jnp.float32)] * 3,
)


@jax.jit
def kernel(x_unsampled, x_sampled):
  qx = x_unsampled[:, :, 0]
  qy = x_unsampled[:, :, 1]
  qz = x_unsampled[:, :, 2]
  kx = x_sampled[:, :, 0]
  ky = x_sampled[:, :, 1]
  kz = x_sampled[:, :, 2]

  sc_out = _sc_knn(qx[:, :NQ_SC], qy[:, :NQ_SC], qz[:, :NQ_SC], kx, ky, kz)

  def tile4(a):
    return a[:, NQ_SC:].reshape(B, NGT, 1, TQ)
  kcol = lambda a: a.reshape(B, M, 1)
  tc_out = _tc_knn(tile4(qx), tile4(qy), tile4(qz), kcol(kx), kcol(ky),
                   kcol(kz))

  parts = []
  for p_sc, p_tc in zip(sc_out, tc_out):
    parts.append(jnp.concatenate((p_sc, p_tc.reshape(B, NQ_TC)), axis=1))
  ia, ib, ic, wa, wb, wc = parts
  idx = jnp.stack((ia, ib, ic), axis=-1)
  w = jnp.stack((wa, wb, wc), axis=-1)[..., None]
  return idx, w


# speedup vs baseline: 64.9961x; 1.1693x over previous
import functools

import jax
import jax.numpy as jnp
from jax import lax
from jax.experimental import pallas as pl
from jax.experimental.pallas import tpu as pltpu
from jax.experimental.pallas import tpu_sc as plsc

B = 4
NQ = 8192
M = 2048
K = 3
L = 16
NC = 2
NS = 16
NW = NC * NS
WPB = NW // B

NQ_SC = 3072
NQ_TC = NQ - NQ_SC
TQ = 256
NGT = NQ_TC // TQ

QPW = NQ_SC // WPB
NG = QPW // L

_EPS2 = 1e-20
_BIGW = 1e10


def _bf16_round(x):
  u = lax.bitcast_convert_type(x, jnp.int32)
  r = (u + 0x7FFF + (lax.shift_right_logical(u, 16) & 1)) & jnp.int32(-65536)
  return lax.bitcast_convert_type(r, jnp.float32)


def _rsqrt_f32(x):
  i = lax.bitcast_convert_type(x, jnp.int32)
  i = 0x5F3759DF - lax.shift_right_logical(i, 1)
  y = lax.bitcast_convert_type(i, jnp.float32)
  for _ in range(3):
    y = y * (1.5 - 0.5 * x * y * y)
  return y


def _tec_body(qx_h, qy_h, qz_h, kx_h, ky_h, kz_h,
              i0_h, i1_h, i2_h, w0_h, w1_h, w2_h,
              qx_v, qy_v, qz_v, kx_v, ky_v, kz_v, h_v,
              i0_s, i1_s, i2_s, w0_s, w1_s, w2_s):
  wid = lax.axis_index("s") * NC + lax.axis_index("c")
  b = wid // WPB
  q0 = (wid % WPB) * QPW

  pltpu.sync_copy(qx_h.at[b, pl.ds(q0, QPW)], qx_v)
  pltpu.sync_copy(qy_h.at[b, pl.ds(q0, QPW)], qy_v)
  pltpu.sync_copy(qz_h.at[b, pl.ds(q0, QPW)], qz_v)
  pltpu.sync_copy(kx_h.at[b], kx_v)
  pltpu.sync_copy(ky_h.at[b], ky_v)
  pltpu.sync_copy(kz_h.at[b], kz_v)

  def h_body(i, carry):
    sl = pl.ds(i * L, L)
    kx = kx_v[sl]
    ky = ky_v[sl]
    kz = kz_v[sl]
    h_v[sl] = kx * kx + ky * ky + kz * kz
    kx_v[sl] = 2.0 * _bf16_round(kx)
    ky_v[sl] = 2.0 * _bf16_round(ky)
    kz_v[sl] = 2.0 * _bf16_round(kz)
    return carry
  lax.fori_loop(0, M // L, h_body, 0)

  lanes = lax.iota(jnp.int32, L)
  inf_v = jnp.full((L,), jnp.inf, jnp.float32)
  zero_i = jnp.zeros((L,), jnp.int32)

  def g_body(g, carry):
    qx = qx_v[pl.ds(g * L, L)]
    qy = qy_v[pl.ds(g * L, L)]
    qz = qz_v[pl.ds(g * L, L)]
    qxr = _bf16_round(qx)
    qyr = _bf16_round(qy)
    qzr = _bf16_round(qz)
    t1 = qx * qx + qy * qy + qz * qz

    def jb_body(jb, c):
      s0, s1, s2, i0, i1, i2 = c
      kxv = kx_v[pl.ds(jb * L, L)]
      kyv = ky_v[pl.ds(jb * L, L)]
      kzv = kz_v[pl.ds(jb * L, L)]
      hv = h_v[pl.ds(jb * L, L)]
      jbase = jb * L
      for m in range(L):
        t3d = qxr * kxv[m] + qyr * kyv[m] + qzr * kzv[m]
        s = jnp.maximum((t1 + hv[m]) - t3d, 0.0)
        jv = zero_i + (jbase + m)
        c0 = s < s0
        c1 = s < s1
        c2 = s < s2
        n1 = jnp.where(c0, s0, jnp.minimum(s, s1))
        m1 = jnp.where(c0, i0, jnp.where(c1, jv, i1))
        s2 = jnp.where(c1, s1, jnp.minimum(s, s2))
        i2 = jnp.where(c1, i1, jnp.where(c2, jv, i2))
        s0 = jnp.minimum(s, s0)
        i0 = jnp.where(c0, jv, i0)
        s1, i1 = n1, m1
      return (s0, s1, s2, i0, i1, i2)

    s0, s1, s2, i0, i1, i2 = lax.fori_loop(
        0, M // L, jb_body, (inf_v, inf_v, inf_v, zero_i, zero_i, zero_i))

    ws = []
    for sm in (s0, s1, s2):
      d2 = jnp.maximum(sm, 0.0)
      w = jnp.where(d2 < _EPS2, _BIGW, _rsqrt_f32(d2))
      ws.append(w)
    wsum = ws[0] + ws[1] + ws[2]

    sl = pl.ds(g * L, L)
    i0_s[sl] = i0
    i1_s[sl] = i1
    i2_s[sl] = i2
    w0_s[sl] = ws[0] / wsum
    w1_s[sl] = ws[1] / wsum
    w2_s[sl] = ws[2] / wsum
    return carry

  lax.fori_loop(0, NG, g_body, 0)

  sl = pl.ds(q0, QPW)
  pltpu.sync_copy(i0_s, i0_h.at[b, sl])
  pltpu.sync_copy(i1_s, i1_h.at[b, sl])
  pltpu.sync_copy(i2_s, i2_h.at[b, sl])
  pltpu.sync_copy(w0_s, w0_h.at[b, sl])
  pltpu.sync_copy(w1_s, w1_h.at[b, sl])
  pltpu.sync_copy(w2_s, w2_h.at[b, sl])


_mesh = plsc.VectorSubcoreMesh(core_axis_name="c", subcore_axis_name="s")

_sc_knn = functools.partial(
    pl.kernel,
    mesh=_mesh,
    out_type=[jax.ShapeDtypeStruct((B, NQ_SC), jnp.int32)] * 3
    + [jax.ShapeDtypeStruct((B, NQ_SC), jnp.float32)] * 3,
    scratch_types=[
        pltpu.VMEM((QPW,), jnp.float32),
        pltpu.VMEM((QPW,), jnp.float32),
        pltpu.VMEM((QPW,), jnp.float32),
        pltpu.VMEM((M,), jnp.float32),
        pltpu.VMEM((M,), jnp.float32),
        pltpu.VMEM((M,), jnp.float32),
        pltpu.VMEM((M,), jnp.float32),
        pltpu.VMEM((QPW,), jnp.int32),
        pltpu.VMEM((QPW,), jnp.int32),
        pltpu.VMEM((QPW,), jnp.int32),
        pltpu.VMEM((QPW,), jnp.float32),
        pltpu.VMEM((QPW,), jnp.float32),
        pltpu.VMEM((QPW,), jnp.float32),
    ],
)(_tec_body)


def _tc_body(qx_r, qy_r, qz_r, kx_r, ky_r, kz_r,
             i0_r, i1_r, i2_r, w0_r, w1_r, w2_r):
  qx = qx_r[0, 0]
  qy = qy_r[0, 0]
  qz = qz_r[0, 0]
  kx = kx_r[0]
  ky = ky_r[0]
  kz = kz_r[0]

  t1 = qx * qx + qy * qy + qz * qz
  t2 = kx * kx + ky * ky + kz * kz
  qxr = _bf16_round(qx)
  qyr = _bf16_round(qy)
  qzr = _bf16_round(qz)
  kxr = 2.0 * _bf16_round(kx)
  kyr = 2.0 * _bf16_round(ky)
  kzr = 2.0 * _bf16_round(kz)

  t3d = qxr * kxr + qyr * kyr + qzr * kzr
  s = jnp.maximum((t1 + t2) - t3d, 0.0)
  iota = lax.broadcasted_iota(jnp.int32, (M, TQ), 0)
  inf = jnp.float32(jnp.inf)

  idxs, d2s = [], []
  for r in range(K):
    mn = jnp.min(s, axis=0, keepdims=True)
    ir = jnp.min(jnp.where(s == mn, iota, M), axis=0, keepdims=True)
    idxs.append(ir)
    d2s.append(mn)
    if r < K - 1:
      s = jnp.where(iota == ir, inf, s)

  ws = [jnp.where(d2 < _EPS2, _BIGW, 1.0 / jnp.sqrt(d2)) for d2 in d2s]
  wsum = ws[0] + ws[1] + ws[2]

  i0_r[0, 0] = idxs[0]
  i1_r[0, 0] = idxs[1]
  i2_r[0, 0] = idxs[2]
  w0_r[0, 0] = ws[0] / wsum
  w1_r[0, 0] = ws[1] / wsum
  w2_r[0, 0] = ws[2] / wsum


_q_spec = pl.BlockSpec((1, 1, 1, TQ), lambda b, t: (b, t, 0, 0))
_k_spec = pl.BlockSpec((1, M, 1), lambda b, t: (b, 0, 0))
_o_spec = pl.BlockSpec((1, 1, 1, TQ), lambda b, t: (b, t, 0, 0))

_tc_knn = pl.pallas_call(
    _tc_body,
    grid=(B, NGT),
    in_specs=[_q_spec] * 3 + [_k_spec] * 3,
    out_specs=[_o_spec] * 6,
    out_shape=[jax.ShapeDtypeStruct((B, NGT, 1, TQ), jnp.int32)] * 3
    + [jax.ShapeDtypeStruct((B, NGT, 1, TQ), jnp.float32)] * 3,
)


@jax.jit
def kernel(x_unsampled, x_sampled):
  qx = x_unsampled[:, :, 0]
  qy = x_unsampled[:, :, 1]
  qz = x_unsampled[:, :, 2]
  kx = x_sampled[:, :, 0]
  ky = x_sampled[:, :, 1]
  kz = x_sampled[:, :, 2]

  sc_out = _sc_knn(qx[:, :NQ_SC], qy[:, :NQ_SC], qz[:, :NQ_SC], kx, ky, kz)

  def tile4(a):
    return a[:, NQ_SC:].reshape(B, NGT, 1, TQ)
  kcol = lambda a: a.reshape(B, M, 1)
  tc_out = _tc_knn(tile4(qx), tile4(qy), tile4(qz), kcol(kx), kcol(ky),
                   kcol(kz))

  parts = []
  for p_sc, p_tc in zip(sc_out, tc_out):
    parts.append(jnp.concatenate((p_sc, p_tc.reshape(B, NQ_TC)), axis=1))
  ia, ib, ic, wa, wb, wc = parts
  idx = jnp.stack((ia, ib, ic), axis=-1)
  w = jnp.stack((wa, wb, wc), axis=-1)[..., None]
  return idx, w


# speedup vs baseline: 66.4818x; 1.0229x over previous
import functools

import jax
import jax.numpy as jnp
from jax import lax
from jax.experimental import pallas as pl
from jax.experimental.pallas import tpu as pltpu
from jax.experimental.pallas import tpu_sc as plsc

B = 4
NQ = 8192
M = 2048
K = 3
L = 16
NC = 2
NS = 16
NW = NC * NS
WPB = NW // B

NQ_SC = 3072
NQ_TC = NQ - NQ_SC
TQ = 256
NGT = NQ_TC // TQ

QPW = NQ_SC // WPB
NG = QPW // L

_EPS2 = 1e-20
_BIGW = 1e10


def _bf16_round(x):
  u = lax.bitcast_convert_type(x, jnp.int32)
  r = (u + 0x7FFF + (lax.shift_right_logical(u, 16) & 1)) & jnp.int32(-65536)
  return lax.bitcast_convert_type(r, jnp.float32)


def _rsqrt_f32(x):
  i = lax.bitcast_convert_type(x, jnp.int32)
  i = 0x5F3759DF - lax.shift_right_logical(i, 1)
  y = lax.bitcast_convert_type(i, jnp.float32)
  for _ in range(3):
    y = y * (1.5 - 0.5 * x * y * y)
  return y


def _tec_body(qx_h, qy_h, qz_h, kx_h, ky_h, kz_h,
              i0_h, i1_h, i2_h, w0_h, w1_h, w2_h,
              qx_v, qy_v, qz_v, kx_v, ky_v, kz_v, h_v,
              i0_s, i1_s, i2_s, w0_s, w1_s, w2_s):
  wid = lax.axis_index("s") * NC + lax.axis_index("c")
  b = wid // WPB
  q0 = (wid % WPB) * QPW

  pltpu.sync_copy(qx_h.at[b, pl.ds(q0, QPW)], qx_v)
  pltpu.sync_copy(qy_h.at[b, pl.ds(q0, QPW)], qy_v)
  pltpu.sync_copy(qz_h.at[b, pl.ds(q0, QPW)], qz_v)
  pltpu.sync_copy(kx_h.at[b], kx_v)
  pltpu.sync_copy(ky_h.at[b], ky_v)
  pltpu.sync_copy(kz_h.at[b], kz_v)

  def h_body(i, carry):
    sl = pl.ds(i * L, L)
    kx = kx_v[sl]
    ky = ky_v[sl]
    kz = kz_v[sl]
    h_v[sl] = kx * kx + ky * ky + kz * kz
    kx_v[sl] = 2.0 * _bf16_round(kx)
    ky_v[sl] = 2.0 * _bf16_round(ky)
    kz_v[sl] = 2.0 * _bf16_round(kz)
    return carry
  lax.fori_loop(0, M // L, h_body, 0)

  lanes = lax.iota(jnp.int32, L)
  inf_v = jnp.full((L,), jnp.inf, jnp.float32)
  zero_i = jnp.zeros((L,), jnp.int32)

  def g_body(g, carry):
    qx = qx_v[pl.ds(g * L, L)]
    qy = qy_v[pl.ds(g * L, L)]
    qz = qz_v[pl.ds(g * L, L)]
    qxr = _bf16_round(qx)
    qyr = _bf16_round(qy)
    qzr = _bf16_round(qz)
    t1 = qx * qx + qy * qy + qz * qz

    def jb_body(jb, c):
      s0, s1, s2, i0, i1, i2 = c
      kxv = kx_v[pl.ds(jb * L, L)]
      kyv = ky_v[pl.ds(jb * L, L)]
      kzv = kz_v[pl.ds(jb * L, L)]
      hv = h_v[pl.ds(jb * L, L)]
      jbase = jb * L
      for m in range(L):
        t3d = qxr * kxv[m] + qyr * kyv[m] + qzr * kzv[m]
        s = jnp.maximum((t1 + hv[m]) - t3d, 0.0)
        jv = zero_i + (jbase + m)
        c0 = s < s0
        c1 = s < s1
        c2 = s < s2
        n1 = jnp.where(c0, s0, jnp.minimum(s, s1))
        m1 = jnp.where(c0, i0, jnp.where(c1, jv, i1))
        s2 = jnp.where(c1, s1, jnp.minimum(s, s2))
        i2 = jnp.where(c1, i1, jnp.where(c2, jv, i2))
        s0 = jnp.minimum(s, s0)
        i0 = jnp.where(c0, jv, i0)
        s1, i1 = n1, m1
      return (s0, s1, s2, i0, i1, i2)

    s0, s1, s2, i0, i1, i2 = lax.fori_loop(
        0, M // L, jb_body, (inf_v, inf_v, inf_v, zero_i, zero_i, zero_i))

    ws = []
    for sm in (s0, s1, s2):
      d2 = jnp.maximum(sm, 0.0)
      w = jnp.where(d2 < _EPS2, _BIGW, _rsqrt_f32(d2))
      ws.append(w)
    wsum = ws[0] + ws[1] + ws[2]

    sl = pl.ds(g * L, L)
    i0_s[sl] = i0
    i1_s[sl] = i1
    i2_s[sl] = i2
    w0_s[sl] = ws[0] / wsum
    w1_s[sl] = ws[1] / wsum
    w2_s[sl] = ws[2] / wsum
    return carry

  lax.fori_loop(0, NG, g_body, 0)

  sl = pl.ds(q0, QPW)
  pltpu.sync_copy(i0_s, i0_h.at[b, sl])
  pltpu.sync_copy(i1_s, i1_h.at[b, sl])
  pltpu.sync_copy(i2_s, i2_h.at[b, sl])
  pltpu.sync_copy(w0_s, w0_h.at[b, sl])
  pltpu.sync_copy(w1_s, w1_h.at[b, sl])
  pltpu.sync_copy(w2_s, w2_h.at[b, sl])


_mesh = plsc.VectorSubcoreMesh(core_axis_name="c", subcore_axis_name="s")

_sc_knn = functools.partial(
    pl.kernel,
    mesh=_mesh,
    out_type=[jax.ShapeDtypeStruct((B, NQ_SC), jnp.int32)] * 3
    + [jax.ShapeDtypeStruct((B, NQ_SC), jnp.float32)] * 3,
    scratch_types=[
        pltpu.VMEM((QPW,), jnp.float32),
        pltpu.VMEM((QPW,), jnp.float32),
        pltpu.VMEM((QPW,), jnp.float32),
        pltpu.VMEM((M,), jnp.float32),
        pltpu.VMEM((M,), jnp.float32),
        pltpu.VMEM((M,), jnp.float32),
        pltpu.VMEM((M,), jnp.float32),
        pltpu.VMEM((QPW,), jnp.int32),
        pltpu.VMEM((QPW,), jnp.int32),
        pltpu.VMEM((QPW,), jnp.int32),
        pltpu.VMEM((QPW,), jnp.float32),
        pltpu.VMEM((QPW,), jnp.float32),
        pltpu.VMEM((QPW,), jnp.float32),
    ],
)(_tec_body)


def _tc_body(qx_r, qy_r, qz_r, kx_r, ky_r, kz_r,
             i0_r, i1_r, i2_r, w0_r, w1_r, w2_r):
  qx = qx_r[0, 0]
  qy = qy_r[0, 0]
  qz = qz_r[0, 0]
  kx = kx_r[0]
  ky = ky_r[0]
  kz = kz_r[0]

  t1 = qx * qx + qy * qy + qz * qz
  t2 = kx * kx + ky * ky + kz * kz
  kmat = jnp.concatenate((
      (2.0 * kx).astype(jnp.bfloat16),
      (2.0 * ky).astype(jnp.bfloat16),
      (2.0 * kz).astype(jnp.bfloat16)), axis=1)
  qmat = jnp.concatenate((
      qx.astype(jnp.bfloat16),
      qy.astype(jnp.bfloat16),
      qz.astype(jnp.bfloat16)), axis=0)
  t3d = jnp.dot(kmat, qmat, preferred_element_type=jnp.float32)
  s = jnp.maximum((t1 + t2) - t3d, 0.0)
  iota = lax.broadcasted_iota(jnp.int32, (M, TQ), 0)
  inf = jnp.float32(jnp.inf)

  idxs, d2s = [], []
  for r in range(K):
    mn = jnp.min(s, axis=0, keepdims=True)
    ir = jnp.min(jnp.where(s == mn, iota, M), axis=0, keepdims=True)
    idxs.append(ir)
    d2s.append(mn)
    if r < K - 1:
      s = jnp.where(iota == ir, inf, s)

  ws = [jnp.where(d2 < _EPS2, _BIGW, 1.0 / jnp.sqrt(d2)) for d2 in d2s]
  wsum = ws[0] + ws[1] + ws[2]

  i0_r[0, 0] = idxs[0]
  i1_r[0, 0] = idxs[1]
  i2_r[0, 0] = idxs[2]
  w0_r[0, 0] = ws[0] / wsum
  w1_r[0, 0] = ws[1] / wsum
  w2_r[0, 0] = ws[2] / wsum


_q_spec = pl.BlockSpec((1, 1, 1, TQ), lambda b, t: (b, t, 0, 0))
_k_spec = pl.BlockSpec((1, M, 1), lambda b, t: (b, 0, 0))
_o_spec = pl.BlockSpec((1, 1, 1, TQ), lambda b, t: (b, t, 0, 0))

_tc_knn = pl.pallas_call(
    _tc_body,
    grid=(B, NGT),
    in_specs=[_q_spec] * 3 + [_k_spec] * 3,
    out_specs=[_o_spec] * 6,
    out_shape=[jax.ShapeDtypeStruct((B, NGT, 1, TQ), jnp.int32)] * 3
    + [jax.ShapeDtypeStruct((B, NGT, 1, TQ), jnp.float32)] * 3,
)


@jax.jit
def kernel(x_unsampled, x_sampled):
  qx = x_unsampled[:, :, 0]
  qy = x_unsampled[:, :, 1]
  qz = x_unsampled[:, :, 2]
  kx = x_sampled[:, :, 0]
  ky = x_sampled[:, :, 1]
  kz = x_sampled[:, :, 2]

  sc_out = _sc_knn(qx[:, :NQ_SC], qy[:, :NQ_SC], qz[:, :NQ_SC], kx, ky, kz)

  def tile4(a):
    return a[:, NQ_SC:].reshape(B, NGT, 1, TQ)
  kcol = lambda a: a.reshape(B, M, 1)
  tc_out = _tc_knn(tile4(qx), tile4(qy), tile4(qz), kcol(kx), kcol(ky),
                   kcol(kz))

  parts = []
  for p_sc, p_tc in zip(sc_out, tc_out):
    parts.append(jnp.concatenate((p_sc, p_tc.reshape(B, NQ_TC)), axis=1))
  ia, ib, ic, wa, wb, wc = parts
  idx = jnp.stack((ia, ib, ic), axis=-1)
  w = jnp.stack((wa, wb, wc), axis=-1)[..., None]
  return idx, w


# speedup vs baseline: 68.9262x; 1.0368x over previous
import functools

import jax
import jax.numpy as jnp
from jax import lax
from jax.experimental import pallas as pl
from jax.experimental.pallas import tpu as pltpu
from jax.experimental.pallas import tpu_sc as plsc

B = 4
NQ = 8192
M = 2048
K = 3
L = 16
NC = 2
NS = 16
NW = NC * NS
WPB = NW // B

NQ_SC = 3072
NQ_TC = NQ - NQ_SC
TQ = 256
NGT = NQ_TC // TQ

QPW = NQ_SC // WPB
NG = QPW // L

_EPS2 = 1e-20
_BIGW = 1e10


def _bf16_round(x):
  u = lax.bitcast_convert_type(x, jnp.int32)
  r = (u + 0x7FFF + (lax.shift_right_logical(u, 16) & 1)) & jnp.int32(-65536)
  return lax.bitcast_convert_type(r, jnp.float32)


def _rsqrt_f32(x):
  i = lax.bitcast_convert_type(x, jnp.int32)
  i = 0x5F3759DF - lax.shift_right_logical(i, 1)
  y = lax.bitcast_convert_type(i, jnp.float32)
  for _ in range(3):
    y = y * (1.5 - 0.5 * x * y * y)
  return y


def _tec_body(qx_h, qy_h, qz_h, kx_h, ky_h, kz_h,
              i0_h, i1_h, i2_h, w0_h, w1_h, w2_h,
              qx_v, qy_v, qz_v, kx_v, ky_v, kz_v, h_v,
              i0_s, i1_s, i2_s, w0_s, w1_s, w2_s):
  wid = lax.axis_index("s") * NC + lax.axis_index("c")
  b = wid // WPB
  q0 = (wid % WPB) * QPW

  pltpu.sync_copy(qx_h.at[b, pl.ds(q0, QPW)], qx_v)
  pltpu.sync_copy(qy_h.at[b, pl.ds(q0, QPW)], qy_v)
  pltpu.sync_copy(qz_h.at[b, pl.ds(q0, QPW)], qz_v)
  pltpu.sync_copy(kx_h.at[b], kx_v)
  pltpu.sync_copy(ky_h.at[b], ky_v)
  pltpu.sync_copy(kz_h.at[b], kz_v)

  def h_body(i, carry):
    sl = pl.ds(i * L, L)
    kx = kx_v[sl]
    ky = ky_v[sl]
    kz = kz_v[sl]
    h_v[sl] = kx * kx + ky * ky + kz * kz
    kx_v[sl] = 2.0 * _bf16_round(kx)
    ky_v[sl] = 2.0 * _bf16_round(ky)
    kz_v[sl] = 2.0 * _bf16_round(kz)
    return carry
  lax.fori_loop(0, M // L, h_body, 0)

  lanes = lax.iota(jnp.int32, L)
  inf_v = jnp.full((L,), jnp.inf, jnp.float32)
  zero_i = jnp.zeros((L,), jnp.int32)

  def g_body(g, carry):
    qx = qx_v[pl.ds(g * L, L)]
    qy = qy_v[pl.ds(g * L, L)]
    qz = qz_v[pl.ds(g * L, L)]
    qxr = _bf16_round(qx)
    qyr = _bf16_round(qy)
    qzr = _bf16_round(qz)
    t1 = qx * qx + qy * qy + qz * qz

    def jb_body(jb, c):
      s0, s1, s2, i0, i1, i2 = c
      base = jb * (2 * L)
      for half in range(2):
        off = base + half * L
        kxv = kx_v[pl.ds(off, L)]
        kyv = ky_v[pl.ds(off, L)]
        kzv = kz_v[pl.ds(off, L)]
        hv = h_v[pl.ds(off, L)]
        for m in range(L):
          t3d = qxr * kxv[m] + qyr * kyv[m] + qzr * kzv[m]
          s = jnp.maximum((t1 + hv[m]) - t3d, 0.0)
          jv = zero_i + (off + m)
          c0 = s < s0
          c1 = s < s1
          c2 = s < s2
          n1 = jnp.where(c0, s0, jnp.minimum(s, s1))
          m1 = jnp.where(c0, i0, jnp.where(c1, jv, i1))
          s2 = jnp.where(c1, s1, jnp.minimum(s, s2))
          i2 = jnp.where(c1, i1, jnp.where(c2, jv, i2))
          s0 = jnp.minimum(s, s0)
          i0 = jnp.where(c0, jv, i0)
          s1, i1 = n1, m1
      return (s0, s1, s2, i0, i1, i2)

    s0, s1, s2, i0, i1, i2 = lax.fori_loop(
        0, M // (2 * L), jb_body,
        (inf_v, inf_v, inf_v, zero_i, zero_i, zero_i))

    ws = []
    for sm in (s0, s1, s2):
      d2 = jnp.maximum(sm, 0.0)
      w = jnp.where(d2 < _EPS2, _BIGW, _rsqrt_f32(d2))
      ws.append(w)
    wsum = ws[0] + ws[1] + ws[2]

    sl = pl.ds(g * L, L)
    i0_s[sl] = i0
    i1_s[sl] = i1
    i2_s[sl] = i2
    w0_s[sl] = ws[0] / wsum
    w1_s[sl] = ws[1] / wsum
    w2_s[sl] = ws[2] / wsum
    return carry

  lax.fori_loop(0, NG, g_body, 0)

  sl = pl.ds(q0, QPW)
  pltpu.sync_copy(i0_s, i0_h.at[b, sl])
  pltpu.sync_copy(i1_s, i1_h.at[b, sl])
  pltpu.sync_copy(i2_s, i2_h.at[b, sl])
  pltpu.sync_copy(w0_s, w0_h.at[b, sl])
  pltpu.sync_copy(w1_s, w1_h.at[b, sl])
  pltpu.sync_copy(w2_s, w2_h.at[b, sl])


_mesh = plsc.VectorSubcoreMesh(core_axis_name="c", subcore_axis_name="s")

_sc_knn = functools.partial(
    pl.kernel,
    mesh=_mesh,
    out_type=[jax.ShapeDtypeStruct((B, NQ_SC), jnp.int32)] * 3
    + [jax.ShapeDtypeStruct((B, NQ_SC), jnp.float32)] * 3,
    scratch_types=[
        pltpu.VMEM((QPW,), jnp.float32),
        pltpu.VMEM((QPW,), jnp.float32),
        pltpu.VMEM((QPW,), jnp.float32),
        pltpu.VMEM((M,), jnp.float32),
        pltpu.VMEM((M,), jnp.float32),
        pltpu.VMEM((M,), jnp.float32),
        pltpu.VMEM((M,), jnp.float32),
        pltpu.VMEM((QPW,), jnp.int32),
        pltpu.VMEM((QPW,), jnp.int32),
        pltpu.VMEM((QPW,), jnp.int32),
        pltpu.VMEM((QPW,), jnp.float32),
        pltpu.VMEM((QPW,), jnp.float32),
        pltpu.VMEM((QPW,), jnp.float32),
    ],
)(_tec_body)


def _tc_body(qx_r, qy_r, qz_r, kx_r, ky_r, kz_r,
             i0_r, i1_r, i2_r, w0_r, w1_r, w2_r):
  qx = qx_r[0, 0]
  qy = qy_r[0, 0]
  qz = qz_r[0, 0]
  kx = kx_r[0]
  ky = ky_r[0]
  kz = kz_r[0]

  t1 = qx * qx + qy * qy + qz * qz
  t2 = kx * kx + ky * ky + kz * kz
  kmat = jnp.concatenate((
      (2.0 * kx).astype(jnp.bfloat16),
      (2.0 * ky).astype(jnp.bfloat16),
      (2.0 * kz).astype(jnp.bfloat16)), axis=1)
  qmat = jnp.concatenate((
      qx.astype(jnp.bfloat16),
      qy.astype(jnp.bfloat16),
      qz.astype(jnp.bfloat16)), axis=0)
  t3d = jnp.dot(kmat, qmat, preferred_element_type=jnp.float32)
  s = jnp.maximum((t1 + t2) - t3d, 0.0)
  iota = lax.broadcasted_iota(jnp.int32, (M, TQ), 0)
  inf = jnp.float32(jnp.inf)

  idxs, d2s = [], []
  for r in range(K):
    mn = jnp.min(s, axis=0, keepdims=True)
    ir = jnp.min(jnp.where(s == mn, iota, M), axis=0, keepdims=True)
    idxs.append(ir)
    d2s.append(mn)
    if r < K - 1:
      s = jnp.where(iota == ir, inf, s)

  ws = [jnp.where(d2 < _EPS2, _BIGW, 1.0 / jnp.sqrt(d2)) for d2 in d2s]
  wsum = ws[0] + ws[1] + ws[2]

  i0_r[0, 0] = idxs[0]
  i1_r[0, 0] = idxs[1]
  i2_r[0, 0] = idxs[2]
  w0_r[0, 0] = ws[0] / wsum
  w1_r[0, 0] = ws[1] / wsum
  w2_r[0, 0] = ws[2] / wsum


_q_spec = pl.BlockSpec((1, 1, 1, TQ), lambda b, t: (b, t, 0, 0))
_k_spec = pl.BlockSpec((1, M, 1), lambda b, t: (b, 0, 0))
_o_spec = pl.BlockSpec((1, 1, 1, TQ), lambda b, t: (b, t, 0, 0))

_tc_knn = pl.pallas_call(
    _tc_body,
    grid=(B, NGT),
    in_specs=[_q_spec] * 3 + [_k_spec] * 3,
    out_specs=[_o_spec] * 6,
    out_shape=[jax.ShapeDtypeStruct((B, NGT, 1, TQ), jnp.int32)] * 3
    + [jax.ShapeDtypeStruct((B, NGT, 1, TQ), jnp.float32)] * 3,
)


@jax.jit
def kernel(x_unsampled, x_sampled):
  qx = x_unsampled[:, :, 0]
  qy = x_unsampled[:, :, 1]
  qz = x_unsampled[:, :, 2]
  kx = x_sampled[:, :, 0]
  ky = x_sampled[:, :, 1]
  kz = x_sampled[:, :, 2]

  sc_out = _sc_knn(qx[:, :NQ_SC], qy[:, :NQ_SC], qz[:, :NQ_SC], kx, ky, kz)

  def tile4(a):
    return a[:, NQ_SC:].reshape(B, NGT, 1, TQ)
  kcol = lambda a: a.reshape(B, M, 1)
  tc_out = _tc_knn(tile4(qx), tile4(qy), tile4(qz), kcol(kx), kcol(ky),
                   kcol(kz))

  parts = []
  for p_sc, p_tc in zip(sc_out, tc_out):
    parts.append(jnp.concatenate((p_sc, p_tc.reshape(B, NQ_TC)), axis=1))
  ia, ib, ic, wa, wb, wc = parts
  idx = jnp.stack((ia, ib, ic), axis=-1)
  w = jnp.stack((wa, wb, wc), axis=-1)[..., None]
  return idx, w
